# baseline (speedup 1.0000x reference)
import functools

import jax
import jax.numpy as jnp
from jax.experimental import pallas as pl

RAYS = 65536
NC = 64
NF = 128
NB = NC - 1
NW = NC - 2
NOUT = NC + NF
NSORT = 256
C = 128


def _substage(x, nrows, j, k, descending=False):
    m = nrows // (2 * j)
    y = x.reshape(m, 2 * j, C)
    a = y[:, :j, :]
    b = y[:, j:, :]
    lo = jnp.minimum(a, b)
    hi = jnp.maximum(a, b)
    if k >= nrows and not descending:
        na, nb = lo, hi
    elif k >= nrows:
        na, nb = hi, lo
    else:
        blk = jax.lax.broadcasted_iota(jnp.int32, (m, 1, C), 0)
        asc = ((blk * (2 * j)) & k) == 0
        if descending:
            asc = jnp.logical_not(asc)
        na = jnp.where(asc, lo, hi)
        nb = jnp.where(asc, hi, lo)
    return jnp.concatenate([na, nb], axis=1).reshape(nrows, C)


def _bitonic_sort(x, nrows, descending=False):
    k = 2
    while k <= nrows:
        j = k // 2
        while j >= 1:
            x = _substage(x, nrows, j, k, descending)
            j //= 2
        k *= 2
    return x


def _body(u_ref, d_ref, w_ref, o_ref):
    d = d_ref[...].T
    u = u_ref[...]
    w = w_ref[...].T[1:NC - 1, :] + 1e-5

    mids = 0.5 * (d[1:, :] + d[:-1, :])
    pdf = w / jnp.sum(w, axis=0, keepdims=True)

    ti = jax.lax.broadcasted_iota(jnp.int32, (NW, NW), 0)
    tj = jax.lax.broadcasted_iota(jnp.int32, (NW, NW), 1)
    tri = (tj <= ti).astype(jnp.float32)
    cdf = jnp.dot(tri, pdf, preferred_element_type=jnp.float32)
    F = jnp.concatenate([jnp.zeros((1, C), jnp.float32), cdf], axis=0)

    fdiff = F[1:, :] - F[:-1, :]
    denom = jnp.where(fdiff < 1e-5, 1.0, fdiff)
    bdiff = mids[1:, :] - mids[:-1, :]
    slope = jnp.concatenate(
        [bdiff / denom, jnp.zeros((1, C), jnp.float32)], axis=0)
    alpha = mids - F * slope
    dalpha = alpha[1:, :] - alpha[:-1, :]
    dslope = slope[1:, :] - slope[:-1, :]

    U = jnp.broadcast_to(u, (NF, C))
    accA = jnp.broadcast_to(alpha[0:1, :], (NF, C))
    accB = jnp.broadcast_to(slope[0:1, :], (NF, C))
    for b in range(1, NB):
        m = (F[b:b + 1, :] <= U).astype(jnp.float32)
        accA = accA + m * dalpha[b - 1:b, :]
        accB = accB + m * dslope[b - 1:b, :]
    samples = accA + U * accB

    s_sorted = _bitonic_sort(samples, NF, descending=False)
    d_sorted = _bitonic_sort(d, NC, descending=True)
    x = jnp.concatenate(
        [s_sorted, jnp.full((NSORT - NOUT, C), jnp.inf, jnp.float32),
         d_sorted], axis=0)
    j = NSORT // 2
    while j >= 1:
        x = _substage(x, NSORT, j, NSORT)
        j //= 2

    o_ref[...] = x[:NOUT, :].T


@jax.jit
def _run(depth, weights, u):
    grid = RAYS // C
    return pl.pallas_call(
        _body,
        grid=(grid,),
        in_specs=[
            pl.BlockSpec((NF, 1), lambda i: (0, 0)),
            pl.BlockSpec((C, NC), lambda i: (i, 0)),
            pl.BlockSpec((C, NC), lambda i: (i, 0)),
        ],
        out_specs=pl.BlockSpec((C, NOUT), lambda i: (i, 0)),
        out_shape=jax.ShapeDtypeStruct((RAYS, NOUT), jnp.float32),
    )(u, depth, weights)


def kernel(depth_rays_values_coarse, coarse_weights, perturb):
    del perturb
    u = jnp.linspace(0.0, 1.0, NF, dtype=jnp.float32).reshape(NF, 1)
    return _run(depth_rays_values_coarse, coarse_weights, u)


# speedup vs baseline: 1.1781x; 1.1781x over previous
import functools

import jax
import jax.numpy as jnp
from jax.experimental import pallas as pl

RAYS = 65536
NC = 64
NF = 128
NB = NC - 1
NW = NC - 2
NOUT = NC + NF
NSORT = 256
C = 128


def _substage(x, nrows, j, k, descending=False):
    m = nrows // (2 * j)
    y = x.reshape(m, 2 * j, C)
    a = y[:, :j, :]
    b = y[:, j:, :]
    lo = jnp.minimum(a, b)
    hi = jnp.maximum(a, b)
    if k >= nrows and not descending:
        na, nb = lo, hi
    elif k >= nrows:
        na, nb = hi, lo
    else:
        blk = jax.lax.broadcasted_iota(jnp.int32, (m, 1, C), 0)
        asc = ((blk * (2 * j)) & k) == 0
        if descending:
            asc = jnp.logical_not(asc)
        na = jnp.where(asc, lo, hi)
        nb = jnp.where(asc, hi, lo)
    return jnp.concatenate([na, nb], axis=1).reshape(nrows, C)


def _bitonic_sort(x, nrows, descending=False):
    k = 2
    while k <= nrows:
        j = k // 2
        while j >= 1:
            x = _substage(x, nrows, j, k, descending)
            j //= 2
        k *= 2
    return x


def _body(u_ref, d_ref, w_ref, o_ref):
    d = d_ref[...]
    u = u_ref[...]
    w = w_ref[1:NC - 1, :] + 1e-5

    mids = 0.5 * (d[1:, :] + d[:-1, :])
    pdf = w / jnp.sum(w, axis=0, keepdims=True)

    ti = jax.lax.broadcasted_iota(jnp.int32, (NW, NW), 0)
    tj = jax.lax.broadcasted_iota(jnp.int32, (NW, NW), 1)
    tri = (tj <= ti).astype(jnp.float32)
    cdf = jnp.dot(tri, pdf, preferred_element_type=jnp.float32)
    F = jnp.concatenate([jnp.zeros((1, C), jnp.float32), cdf], axis=0)

    fdiff = F[1:, :] - F[:-1, :]
    denom = jnp.where(fdiff < 1e-5, 1.0, fdiff)
    bdiff = mids[1:, :] - mids[:-1, :]
    slope = jnp.concatenate(
        [bdiff / denom, jnp.zeros((1, C), jnp.float32)], axis=0)
    alpha = mids - F * slope
    dalpha = alpha[1:, :] - alpha[:-1, :]
    dslope = slope[1:, :] - slope[:-1, :]

    U = jnp.broadcast_to(u, (NF, C))
    acc = alpha[0:1, :] + U * slope[0:1, :]
    for b in range(1, NB):
        t = dalpha[b - 1:b, :] + U * dslope[b - 1:b, :]
        m = F[b:b + 1, :] <= U
        acc = acc + jnp.where(m, t, 0.0)
    samples = acc

    s_sorted = _bitonic_sort(samples, NF, descending=False)
    d_sorted = _bitonic_sort(d, NC, descending=True)
    x = jnp.concatenate(
        [s_sorted, jnp.full((NSORT - NOUT, C), jnp.inf, jnp.float32),
         d_sorted], axis=0)
    j = NSORT // 2
    while j >= 1:
        x = _substage(x, NSORT, j, NSORT)
        j //= 2

    o_ref[...] = x[:NOUT, :]


@jax.jit
def _run(depth_t, weights_t, u):
    grid = RAYS // C
    return pl.pallas_call(
        _body,
        grid=(grid,),
        in_specs=[
            pl.BlockSpec((NF, 1), lambda i: (0, 0)),
            pl.BlockSpec((NC, C), lambda i: (0, i)),
            pl.BlockSpec((NC, C), lambda i: (0, i)),
        ],
        out_specs=pl.BlockSpec((NOUT, C), lambda i: (0, i)),
        out_shape=jax.ShapeDtypeStruct((NOUT, RAYS), jnp.float32),
    )(u, depth_t, weights_t)


def kernel(depth_rays_values_coarse, coarse_weights, perturb):
    del perturb
    u = jnp.linspace(0.0, 1.0, NF, dtype=jnp.float32).reshape(NF, 1)
    out_t = _run(depth_rays_values_coarse.T, coarse_weights.T, u)
    return out_t.T
